# baseline (speedup 1.0000x reference)
import functools

import jax
import jax.numpy as jnp
from jax import lax
from jax.experimental import pallas as pl
from jax.experimental.pallas import tpu as pltpu
from jax.experimental.pallas import tpu_sc as plsc

V_DIM = 1000000
EMB_DIM = 64
BATCH = 16384

NC = 2
NS = 16
NW = NC * NS
B_PER_W = BATCH // NW
NBUF = 12
D = 11
L = 16
NG = B_PER_W // L
PADW = 1 + B_PER_W + L
FR = 64
GPF = FR // L
NFL = B_PER_W // FR
KC = 128
NKC = B_PER_W // KC


def _gather_kernel(svp_hbm, tableT_hbm, out_hbm,
                   sv_v, fl_v, sl_v, cols_v, outS_v, sems, sem_out):
    wid = lax.axis_index("s") * NC + lax.axis_index("c")
    base = wid * B_PER_W
    pltpu.sync_copy(svp_hbm.at[wid], sv_v)

    fl_v[pl.ds(B_PER_W, L)] = jnp.zeros((L,), jnp.int32)
    carry = jnp.int32(0)
    for g in range(NG):
        v = sv_v[pl.ds(g * L + 1, L)]
        tc = lax.shift_right_logical(v, 7)
        vm = sv_v[pl.ds(g * L, L)]
        tm = lax.shift_right_logical(vm, 7)
        neq = (tc != tm).astype(jnp.int32)
        if g == 0:
            neq = jnp.where(lax.iota(jnp.int32, L) == 0, 1, neq)
        cum = plsc.cumsum(neq)
        fo = cum + carry - 1
        carry = carry + cum[L - 1]
        slot = fo - (fo // NBUF) * NBUF
        fl_v[pl.ds(g * L, L)] = neq
        sl_v[pl.ds(g * L, L)] = slot

    def fire(x, slot):
        tc = lax.shift_right_logical(x, 7)
        off = pl.multiple_of(tc * 128, 128)
        pltpu.async_copy(
            tableT_hbm.at[:, pl.ds(off, 128)], cols_v.at[slot], sems.at[slot]
        )

    def wait(slot):
        pltpu.make_async_copy(
            tableT_hbm.at[:, pl.ds(0, 128)], cols_v.at[slot], sems.at[slot]
        ).wait()

    def wait_flush(half):
        pltpu.make_async_copy(
            outS_v.at[half], out_hbm.at[pl.ds(base, FR)], sem_out
        ).wait()

    def select(x, slot, i):
        c = jnp.broadcast_to(jnp.bitwise_and(x, 127), (L,))
        sl = jnp.broadcast_to(slot, (L,))
        half = jnp.bitwise_and(i // FR, 1)
        row = jnp.bitwise_and(i, FR - 1)
        for g in range(EMB_DIM // L):
            idx0 = lax.iota(jnp.int32, L) + g * L
            outS_v[half, row, pl.ds(g * L, L)] = plsc.load_gather(
                cols_v, [sl, idx0, c]
            )

    sv0 = sv_v[pl.ds(1, L)]
    fl0 = fl_v[pl.ds(0, L)]
    sl0 = sl_v[pl.ds(0, L)]
    for b in range(D):

        @pl.when(fl0[b] == 1)
        def _():
            fire(sv0[b], sl0[b])

    def body(g, _):
        @pl.when(jnp.bitwise_and(g, GPF - 1) == 0)
        def _():
            fl = g // GPF

            @pl.when(fl >= 2)
            def _():
                wait_flush(jnp.bitwise_and(fl, 1))

        svc = sv_v[pl.ds(g * L + 1, L)]
        flc = fl_v[pl.ds(g * L, L)]
        slc = sl_v[pl.ds(g * L, L)]
        sva = sv_v[pl.ds(g * L + D + 1, L)]
        fla = fl_v[pl.ds(g * L + D, L)]
        sla = sl_v[pl.ds(g * L + D, L)]
        for b in range(L):
            i = g * L + b

            @pl.when(fla[b] == 1)
            def _():
                fire(sva[b], sla[b])

            @pl.when(flc[b] == 1)
            def _():
                wait(slc[b])

            select(svc[b], slc[b], i)

        @pl.when(jnp.bitwise_and(g, GPF - 1) == GPF - 1)
        def _():
            fl = g // GPF
            half = jnp.bitwise_and(fl, 1)
            pltpu.async_copy(
                outS_v.at[half], out_hbm.at[pl.ds(base + fl * FR, FR)],
                sem_out,
            )

        return _

    lax.fori_loop(0, NG, body, None, unroll=False)
    wait_flush(0)
    wait_flush(1)


def _perm_kernel(perm_hbm, rows_hbm, out_hbm, pm_v, rows_v, sem):
    wid = lax.axis_index("s") * NC + lax.axis_index("c")
    base = wid * B_PER_W
    pltpu.sync_copy(perm_hbm.at[wid], pm_v)
    pltpu.sync_copy(rows_hbm.at[pl.ds(base, B_PER_W)], rows_v)
    copies = []
    for j in range(NKC):
        copies.append(
            pltpu.async_copy(
                rows_v.at[pl.ds(j * KC, KC)],
                out_hbm.at[pm_v.at[j]],
                sem,
            )
        )
    for c in copies:
        c.wait()


@jax.jit
def _emb_lookup(svp, perm3, tableT):
    mesh = plsc.VectorSubcoreMesh(
        core_axis_name="c", subcore_axis_name="s", num_cores=NC, num_subcores=NS
    )
    sorted_rows = pl.kernel(
        _gather_kernel,
        out_type=jax.ShapeDtypeStruct((BATCH, EMB_DIM), jnp.float32),
        mesh=mesh,
        scratch_types=[
            pltpu.VMEM((PADW,), jnp.int32),
            pltpu.VMEM((PADW,), jnp.int32),
            pltpu.VMEM((PADW,), jnp.int32),
            pltpu.VMEM((NBUF, EMB_DIM, 128), jnp.float32),
            pltpu.VMEM((2, FR, EMB_DIM), jnp.float32),
            pltpu.SemaphoreType.DMA((NBUF,)),
            pltpu.SemaphoreType.DMA,
        ],
        compiler_params=pltpu.CompilerParams(needs_layout_passes=False),
    )(svp, tableT)
    return pl.kernel(
        _perm_kernel,
        out_type=jax.ShapeDtypeStruct((BATCH, EMB_DIM), jnp.float32),
        mesh=mesh,
        scratch_types=[
            pltpu.VMEM((NKC, KC), jnp.int32),
            pltpu.VMEM((B_PER_W, EMB_DIM), jnp.float32),
            pltpu.SemaphoreType.DMA,
        ],
        compiler_params=pltpu.CompilerParams(
            use_tc_tiling_on_sc=False, needs_layout_passes=False
        ),
    )(perm3, sorted_rows)


def kernel(x, embeddings_weight):
    xi = x.astype(jnp.int32)
    iota = lax.iota(jnp.int32, BATCH)
    sv, perm = lax.sort_key_val(xi, iota)
    sv2 = sv.reshape(NW, B_PER_W)
    svp = jnp.concatenate(
        [sv2[:, :1], sv2, jnp.tile(sv2[:, -1:], (1, L))], axis=1
    )
    perm3 = perm.reshape(NW, NKC, KC)
    tableT = embeddings_weight.T
    return _emb_lookup(svp, perm3, tableT)


# speedup vs baseline: 1.0374x; 1.0374x over previous
import functools

import jax
import jax.numpy as jnp
from jax import lax
from jax.experimental import pallas as pl
from jax.experimental.pallas import tpu as pltpu
from jax.experimental.pallas import tpu_sc as plsc

V_DIM = 1000000
EMB_DIM = 64
BATCH = 16384

NC = 2
NS = 16
NW = NC * NS
B_PER_W = BATCH // NW
NBUF = 14
D = 13
L = 16
NG = B_PER_W // L
PADW = 1 + B_PER_W + L
FR = 32
GPF = FR // L
NFL = B_PER_W // FR
KC = 128
NKC = B_PER_W // KC


def _gather_kernel(svp_hbm, tableT_hbm, out_hbm,
                   sv_v, fl_v, sl_v, cols_v, outS_v, sems, sem_out):
    wid = lax.axis_index("s") * NC + lax.axis_index("c")
    base = wid * B_PER_W
    pltpu.sync_copy(svp_hbm.at[wid], sv_v)

    fl_v[pl.ds(B_PER_W, L)] = jnp.zeros((L,), jnp.int32)
    carry = jnp.int32(0)
    for g in range(NG):
        v = sv_v[pl.ds(g * L + 1, L)]
        tc = lax.shift_right_logical(v, 7)
        vm = sv_v[pl.ds(g * L, L)]
        tm = lax.shift_right_logical(vm, 7)
        neq = (tc != tm).astype(jnp.int32)
        if g == 0:
            neq = jnp.where(lax.iota(jnp.int32, L) == 0, 1, neq)
        cum = plsc.cumsum(neq)
        fo = cum + carry - 1
        carry = carry + cum[L - 1]
        slot = fo - (fo // NBUF) * NBUF
        fl_v[pl.ds(g * L, L)] = neq
        sl_v[pl.ds(g * L, L)] = slot

    def fire(x, slot):
        tc = lax.shift_right_logical(x, 7)
        off = pl.multiple_of(tc * 128, 128)
        pltpu.async_copy(
            tableT_hbm.at[:, pl.ds(off, 128)], cols_v.at[slot], sems.at[slot]
        )

    def wait(slot):
        pltpu.make_async_copy(
            tableT_hbm.at[:, pl.ds(0, 128)], cols_v.at[slot], sems.at[slot]
        ).wait()

    def wait_flush(half):
        pltpu.make_async_copy(
            outS_v.at[half], out_hbm.at[pl.ds(base, FR)], sem_out
        ).wait()

    def select(x, slot, i):
        c = jnp.broadcast_to(jnp.bitwise_and(x, 127), (L,))
        sl = jnp.broadcast_to(slot, (L,))
        half = jnp.bitwise_and(i // FR, 1)
        row = jnp.bitwise_and(i, FR - 1)
        for g in range(EMB_DIM // L):
            idx0 = lax.iota(jnp.int32, L) + g * L
            outS_v[half, row, pl.ds(g * L, L)] = plsc.load_gather(
                cols_v, [sl, idx0, c]
            )

    sv0 = sv_v[pl.ds(1, L)]
    fl0 = fl_v[pl.ds(0, L)]
    sl0 = sl_v[pl.ds(0, L)]
    for b in range(D):

        @pl.when(fl0[b] == 1)
        def _():
            fire(sv0[b], sl0[b])

    def body(g, _):
        @pl.when(jnp.bitwise_and(g, GPF - 1) == 0)
        def _():
            fl = g // GPF

            @pl.when(fl >= 2)
            def _():
                wait_flush(jnp.bitwise_and(fl, 1))

        svc = sv_v[pl.ds(g * L + 1, L)]
        flc = fl_v[pl.ds(g * L, L)]
        slc = sl_v[pl.ds(g * L, L)]
        sva = sv_v[pl.ds(g * L + D + 1, L)]
        fla = fl_v[pl.ds(g * L + D, L)]
        sla = sl_v[pl.ds(g * L + D, L)]
        for b in range(L):
            i = g * L + b

            @pl.when(fla[b] == 1)
            def _():
                fire(sva[b], sla[b])

            @pl.when(flc[b] == 1)
            def _():
                wait(slc[b])

            select(svc[b], slc[b], i)

        @pl.when(jnp.bitwise_and(g, GPF - 1) == GPF - 1)
        def _():
            fl = g // GPF
            half = jnp.bitwise_and(fl, 1)
            pltpu.async_copy(
                outS_v.at[half], out_hbm.at[pl.ds(base + fl * FR, FR)],
                sem_out,
            )

        return _

    lax.fori_loop(0, NG, body, None, unroll=False)
    wait_flush(0)
    wait_flush(1)


def _perm_kernel(perm_hbm, rows_hbm, out_hbm, pm_v, rows_v, sem):
    wid = lax.axis_index("s") * NC + lax.axis_index("c")
    base = wid * B_PER_W
    pltpu.sync_copy(perm_hbm.at[wid], pm_v)
    pltpu.sync_copy(rows_hbm.at[pl.ds(base, B_PER_W)], rows_v)
    copies = []
    for j in range(NKC):
        copies.append(
            pltpu.async_copy(
                rows_v.at[pl.ds(j * KC, KC)],
                out_hbm.at[pm_v.at[j]],
                sem,
            )
        )
    for c in copies:
        c.wait()


@jax.jit
def _emb_lookup(svp, perm3, tableT):
    mesh = plsc.VectorSubcoreMesh(
        core_axis_name="c", subcore_axis_name="s", num_cores=NC, num_subcores=NS
    )
    sorted_rows = pl.kernel(
        _gather_kernel,
        out_type=jax.ShapeDtypeStruct((BATCH, EMB_DIM), jnp.float32),
        mesh=mesh,
        scratch_types=[
            pltpu.VMEM((PADW,), jnp.int32),
            pltpu.VMEM((PADW,), jnp.int32),
            pltpu.VMEM((PADW,), jnp.int32),
            pltpu.VMEM((NBUF, EMB_DIM, 128), jnp.float32),
            pltpu.VMEM((2, FR, EMB_DIM), jnp.float32),
            pltpu.SemaphoreType.DMA((NBUF,)),
            pltpu.SemaphoreType.DMA,
        ],
        compiler_params=pltpu.CompilerParams(needs_layout_passes=False),
    )(svp, tableT)
    return pl.kernel(
        _perm_kernel,
        out_type=jax.ShapeDtypeStruct((BATCH, EMB_DIM), jnp.float32),
        mesh=mesh,
        scratch_types=[
            pltpu.VMEM((NKC, KC), jnp.int32),
            pltpu.VMEM((B_PER_W, EMB_DIM), jnp.float32),
            pltpu.SemaphoreType.DMA,
        ],
        compiler_params=pltpu.CompilerParams(
            use_tc_tiling_on_sc=False, needs_layout_passes=False
        ),
    )(perm3, sorted_rows)


def kernel(x, embeddings_weight):
    xi = x.astype(jnp.int32)
    iota = lax.iota(jnp.int32, BATCH)
    sv, perm = lax.sort_key_val(xi, iota)
    sv2 = sv.reshape(NW, B_PER_W)
    svp = jnp.concatenate(
        [sv2[:, :1], sv2, jnp.tile(sv2[:, -1:], (1, L))], axis=1
    )
    perm3 = perm.reshape(NW, NKC, KC)
    tableT = embeddings_weight.T
    return _emb_lookup(svp, perm3, tableT)


# speedup vs baseline: 1.0422x; 1.0046x over previous
import functools

import jax
import jax.numpy as jnp
from jax import lax
from jax.experimental import pallas as pl
from jax.experimental.pallas import tpu as pltpu
from jax.experimental.pallas import tpu_sc as plsc

V_DIM = 1000000
EMB_DIM = 64
BATCH = 16384

NC = 2
NS = 16
NW = NC * NS
B_PER_W = BATCH // NW
NBUF = 14
D = 13
L = 16
NG = B_PER_W // L
PADW = 1 + B_PER_W + L
FR = 32
GPF = FR // L
NFL = B_PER_W // FR
KC = 128
NKC = B_PER_W // KC


def _gather_kernel(svp_hbm, tableT_hbm, out_hbm,
                   sv_v, fl_v, sl_v, cols_v, outS_v, sems, sem_out):
    wid = lax.axis_index("s") * NC + lax.axis_index("c")
    base = wid * B_PER_W
    pltpu.sync_copy(svp_hbm.at[wid], sv_v)

    fl_v[pl.ds(B_PER_W, L)] = jnp.zeros((L,), jnp.int32)
    carry = jnp.int32(0)
    for g in range(NG):
        v = sv_v[pl.ds(g * L + 1, L)]
        tc = lax.shift_right_logical(v, 7)
        vm = sv_v[pl.ds(g * L, L)]
        tm = lax.shift_right_logical(vm, 7)
        neq = (tc != tm).astype(jnp.int32)
        if g == 0:
            neq = jnp.where(lax.iota(jnp.int32, L) == 0, 1, neq)
        cum = plsc.cumsum(neq)
        fo = cum + carry - 1
        carry = carry + cum[L - 1]
        slot = fo - (fo // NBUF) * NBUF
        fl_v[pl.ds(g * L, L)] = neq
        sl_v[pl.ds(g * L, L)] = slot

    def fire(x, slot):
        tc = lax.shift_right_logical(x, 7)
        off = pl.multiple_of(tc * 128, 128)
        pltpu.async_copy(
            tableT_hbm.at[:, pl.ds(off, 128)], cols_v.at[slot], sems.at[slot]
        )

    def wait(slot):
        pltpu.make_async_copy(
            tableT_hbm.at[:, pl.ds(0, 128)], cols_v.at[slot], sems.at[slot]
        ).wait()

    def wait_flush(half):
        pltpu.make_async_copy(
            outS_v.at[half], out_hbm.at[pl.ds(base, FR)], sem_out
        ).wait()

    def select(x, slot, i):
        c = jnp.broadcast_to(jnp.bitwise_and(x, 127), (L,))
        sl = jnp.broadcast_to(slot, (L,))
        half = jnp.bitwise_and(i // FR, 1)
        row = jnp.bitwise_and(i, FR - 1)
        for g in range(EMB_DIM // L):
            idx0 = lax.iota(jnp.int32, L) + g * L
            outS_v[half, row, pl.ds(g * L, L)] = plsc.load_gather(
                cols_v, [sl, idx0, c]
            )

    sv0 = sv_v[pl.ds(1, L)]
    fl0 = fl_v[pl.ds(0, L)]
    sl0 = sl_v[pl.ds(0, L)]
    for b in range(D):

        @pl.when(fl0[b] == 1)
        def _():
            fire(sv0[b], sl0[b])

    def body(g, _):
        @pl.when(jnp.bitwise_and(g, GPF - 1) == 0)
        def _():
            fl = g // GPF

            @pl.when(fl >= 2)
            def _():
                wait_flush(jnp.bitwise_and(fl, 1))

        svc = sv_v[pl.ds(g * L + 1, L)]
        flc = fl_v[pl.ds(g * L, L)]
        slc = sl_v[pl.ds(g * L, L)]
        sva = sv_v[pl.ds(g * L + D + 1, L)]
        fla = fl_v[pl.ds(g * L + D, L)]
        sla = sl_v[pl.ds(g * L + D, L)]
        for b in range(L):
            i = g * L + b

            @pl.when(fla[b] == 1)
            def _():
                fire(sva[b], sla[b])

            @pl.when(flc[b] == 1)
            def _():
                wait(slc[b])

            select(svc[b], slc[b], i)

        @pl.when(jnp.bitwise_and(g, GPF - 1) == GPF - 1)
        def _():
            fl = g // GPF
            half = jnp.bitwise_and(fl, 1)
            pltpu.async_copy(
                outS_v.at[half], out_hbm.at[pl.ds(base + fl * FR, FR)],
                sem_out,
            )

        return _

    lax.fori_loop(0, NG, body, None, unroll=False)
    wait_flush(0)
    wait_flush(1)


def _perm_kernel(perm_hbm, rows_hbm, out_hbm, pm_v, rows_v, sem):
    wid = lax.axis_index("s") * NC + lax.axis_index("c")
    base = wid * B_PER_W
    pltpu.sync_copy(perm_hbm.at[wid], pm_v)
    pltpu.sync_copy(rows_hbm.at[pl.ds(base, B_PER_W)], rows_v)
    copies = []
    for j in range(NKC):
        copies.append(
            pltpu.async_copy(
                rows_v.at[pl.ds(j * KC, KC)],
                out_hbm.at[pm_v.at[j]],
                sem,
            )
        )
    for c in copies:
        c.wait()


@jax.jit
def _emb_lookup(svp, perm3, tableT):
    mesh = plsc.VectorSubcoreMesh(
        core_axis_name="c", subcore_axis_name="s", num_cores=NC, num_subcores=NS
    )
    sorted_rows = pl.kernel(
        _gather_kernel,
        out_type=jax.ShapeDtypeStruct((BATCH, EMB_DIM), jnp.float32),
        mesh=mesh,
        scratch_types=[
            pltpu.VMEM((PADW,), jnp.int32),
            pltpu.VMEM((PADW,), jnp.int32),
            pltpu.VMEM((PADW,), jnp.int32),
            pltpu.VMEM((NBUF, EMB_DIM, 128), jnp.float32),
            pltpu.VMEM((2, FR, EMB_DIM), jnp.float32),
            pltpu.SemaphoreType.DMA((NBUF,)),
            pltpu.SemaphoreType.DMA,
        ],
        compiler_params=pltpu.CompilerParams(needs_layout_passes=False),
    )(svp, tableT)
    return pl.kernel(
        _perm_kernel,
        out_type=jax.ShapeDtypeStruct((BATCH, EMB_DIM), jnp.float32),
        mesh=mesh,
        scratch_types=[
            pltpu.VMEM((NKC, KC), jnp.int32),
            pltpu.VMEM((B_PER_W, EMB_DIM), jnp.float32),
            pltpu.SemaphoreType.DMA,
        ],
        compiler_params=pltpu.CompilerParams(
            use_tc_tiling_on_sc=False, needs_layout_passes=False
        ),
    )(perm3, sorted_rows)


def kernel(x, embeddings_weight):
    xi = x.astype(jnp.int32)
    iota = lax.iota(jnp.int32, BATCH)
    sv, perm = lax.sort((xi, iota), num_keys=1, is_stable=False)
    sv2 = sv.reshape(NW, B_PER_W)
    svp = jnp.concatenate(
        [sv2[:, :1], sv2, jnp.tile(sv2[:, -1:], (1, L))], axis=1
    )
    perm3 = perm.reshape(NW, NKC, KC)
    tableT = embeddings_weight.T
    return _emb_lookup(svp, perm3, tableT)
